# baseline (speedup 1.0000x reference)
import functools

import jax
import jax.numpy as jnp
from jax import lax
from jax.experimental import pallas as pl
from jax.experimental.pallas import tpu as pltpu
from jax.experimental.pallas import tpu_sc as plsc

_NS = 4
_LG = 8
_GW = 256
_ROWS = 8 * 2048
_COLS = _LG * _GW
_OUTW = _NS * _LG

_NC = 2
_NSUB = 16
_NW = _NC * _NSUB
_ROWS_PER_W = _ROWS // _NW
_CHUNK = 16
_NCHUNK = _ROWS_PER_W // _CHUNK
_ILP = 4
_SPAN = _GW // _ILP


def _insert(m, x):
    m0, m1, m2, m3 = m
    t0 = jnp.minimum(m0, x)
    m0 = jnp.maximum(m0, x)
    t1 = jnp.minimum(m1, t0)
    m1 = jnp.maximum(m1, t0)
    t2 = jnp.minimum(m2, t1)
    m2 = jnp.maximum(m2, t1)
    m3 = jnp.maximum(m3, t2)
    return (m0, m1, m2, m3)


def _merge4(a, b):
    a0, a1, a2, a3 = a
    b0, b1, b2, b3 = b
    c0 = jnp.maximum(a0, b0)
    c1 = jnp.maximum(jnp.maximum(a1, b1), jnp.minimum(a0, b0))
    c2 = jnp.maximum(
        jnp.maximum(a2, b2),
        jnp.maximum(jnp.minimum(a1, b0), jnp.minimum(a0, b1)),
    )
    c3 = jnp.maximum(
        jnp.maximum(a3, b3),
        jnp.maximum(
            jnp.minimum(a2, b0),
            jnp.maximum(jnp.minimum(a1, b1), jnp.minimum(a0, b2)),
        ),
    )
    return (c0, c1, c2, c3)


_UNROLL = 8


_WPB = 2048 // _ROWS_PER_W


def _body(in_hbm, out_hbm, buf0, buf1, obuf, sem0, sem1):
    wid = lax.axis_index("s") * _NC + lax.axis_index("c")
    b = wid // _WPB
    r0 = (wid % _WPB) * _ROWS_PER_W
    rows_iota = lax.broadcasted_iota(jnp.int32, (16,), 0)
    neg = jnp.full((16,), -jnp.inf, jnp.float32)

    def in_slice(c):
        return in_hbm.at[b, pl.ds(r0 + c * _CHUNK, _CHUNK)]

    def compute_chunk(buf, c):
        orow = c * _CHUNK
        for g in range(_LG):
            g0 = g * _GW

            def elem_body(j, ms):
                off = (rows_iota ^ j) & (_SPAN - 1)
                col = g0 + off
                out = []
                for s in range(_ILP):
                    x = plsc.load_gather(buf, [rows_iota, col + (s * _SPAN)])
                    out.append(_insert(ms[s], x))
                return tuple(out)

            init = tuple((neg, neg, neg, neg) for _ in range(_ILP))
            ms = plsc.parallel_loop(
                0, _SPAN, 1, unroll=_UNROLL, carry=init)(elem_body)
            ms = list(ms)
            while len(ms) > 1:
                ms = [_merge4(ms[i], ms[i + 1])
                      for i in range(0, len(ms), 2)]
            m = ms[0]
            for k in range(_NS):
                plsc.store_scatter(
                    obuf,
                    [(rows_iota + orow) * _OUTW + (g * _NS + k)],
                    m[k])

    pltpu.async_copy(in_slice(0), buf0, sem0)

    def loop_body(t, carry):
        c0 = 2 * t
        pltpu.async_copy(in_slice(c0 + 1), buf1, sem1)
        pltpu.make_async_copy(in_slice(c0), buf0, sem0).wait()
        compute_chunk(buf0, c0)

        @pl.when(t < _NCHUNK // 2 - 1)
        def _():
            pltpu.async_copy(in_slice(c0 + 2), buf0, sem0)

        pltpu.make_async_copy(in_slice(c0 + 1), buf1, sem1).wait()
        compute_chunk(buf1, c0 + 1)
        return carry

    lax.fori_loop(0, _NCHUNK // 2, loop_body, 0)
    row0 = b * 2048 + r0
    pltpu.sync_copy(
        obuf, out_hbm.at[pl.ds(row0 * _OUTW, _ROWS_PER_W * _OUTW)])


@functools.cache
def _sc_call():
    return pl.kernel(
        _body,
        out_type=jax.ShapeDtypeStruct((_ROWS * _OUTW,), jnp.float32),
        mesh=plsc.VectorSubcoreMesh(
            core_axis_name="c", subcore_axis_name="s",
            num_cores=_NC, num_subcores=_NSUB,
        ),
        scratch_types=[
            pltpu.VMEM((_CHUNK, _COLS), jnp.float32),
            pltpu.VMEM((_CHUNK, _COLS), jnp.float32),
            pltpu.VMEM((_ROWS_PER_W * _OUTW,), jnp.float32),
            pltpu.SemaphoreType.DMA,
            pltpu.SemaphoreType.DMA,
        ],
        compiler_params=pltpu.CompilerParams(
            use_tc_tiling_on_sc=True, needs_layout_passes=False),
    )


@jax.jit
def kernel(inputs):
    y = _sc_call()(inputs)
    return y.reshape(8, 2048, _OUTW)


# speedup vs baseline: 1.0795x; 1.0795x over previous
import functools

import jax
import jax.numpy as jnp
from jax import lax
from jax.experimental import pallas as pl
from jax.experimental.pallas import tpu as pltpu
from jax.experimental.pallas import tpu_sc as plsc

_NS = 4
_LG = 8
_GW = 256
_ROWS = 8 * 2048
_COLS = _LG * _GW
_OUTW = _NS * _LG

_NC = 2
_NSUB = 16
_NW = _NC * _NSUB
_ROWS_PER_W = _ROWS // _NW
_CHUNK = 16
_NCHUNK = _ROWS_PER_W // _CHUNK
_ILP = 4
_SPAN = _GW // _ILP


def _insert(m, x):
    m0, m1, m2, m3 = m
    t0 = jnp.minimum(m0, x)
    m0 = jnp.maximum(m0, x)
    t1 = jnp.minimum(m1, t0)
    m1 = jnp.maximum(m1, t0)
    t2 = jnp.minimum(m2, t1)
    m2 = jnp.maximum(m2, t1)
    m3 = jnp.maximum(m3, t2)
    return (m0, m1, m2, m3)


def _merge4(a, b):
    a0, a1, a2, a3 = a
    b0, b1, b2, b3 = b
    c0 = jnp.maximum(a0, b0)
    c1 = jnp.maximum(jnp.maximum(a1, b1), jnp.minimum(a0, b0))
    c2 = jnp.maximum(
        jnp.maximum(a2, b2),
        jnp.maximum(jnp.minimum(a1, b0), jnp.minimum(a0, b1)),
    )
    c3 = jnp.maximum(
        jnp.maximum(a3, b3),
        jnp.maximum(
            jnp.minimum(a2, b0),
            jnp.maximum(jnp.minimum(a1, b1), jnp.minimum(a0, b2)),
        ),
    )
    return (c0, c1, c2, c3)


_UNROLL = 4


_WPB = 2048 // _ROWS_PER_W


def _body(in_hbm, out_hbm, buf0, buf1, obuf, sem0, sem1):
    wid = lax.axis_index("s") * _NC + lax.axis_index("c")
    b = wid // _WPB
    r0 = (wid % _WPB) * _ROWS_PER_W
    rows_iota = lax.broadcasted_iota(jnp.int32, (16,), 0)
    neg = jnp.full((16,), -jnp.inf, jnp.float32)

    def in_slice(c):
        return in_hbm.at[b, pl.ds(r0 + c * _CHUNK, _CHUNK)]

    def compute_chunk(buf, c):
        orow = c * _CHUNK
        for g in range(_LG):
            g0 = g * _GW

            def elem_body(j, ms):
                off = (rows_iota ^ j) & (_SPAN - 1)
                col = g0 + off
                out = []
                for s in range(_ILP):
                    x = plsc.load_gather(buf, [rows_iota, col + (s * _SPAN)])
                    out.append(_insert(ms[s], x))
                return tuple(out)

            init = tuple((neg, neg, neg, neg) for _ in range(_ILP))
            ms = plsc.parallel_loop(
                0, _SPAN, 1, unroll=_UNROLL, carry=init)(elem_body)
            ms = list(ms)
            while len(ms) > 1:
                ms = [_merge4(ms[i], ms[i + 1])
                      for i in range(0, len(ms), 2)]
            m = ms[0]
            for k in range(_NS):
                plsc.store_scatter(
                    obuf,
                    [(rows_iota + orow) * _OUTW + (g * _NS + k)],
                    m[k])

    pltpu.async_copy(in_slice(0), buf0, sem0)

    def loop_body(t, carry):
        c0 = 2 * t
        pltpu.async_copy(in_slice(c0 + 1), buf1, sem1)
        pltpu.make_async_copy(in_slice(c0), buf0, sem0).wait()
        compute_chunk(buf0, c0)

        @pl.when(t < _NCHUNK // 2 - 1)
        def _():
            pltpu.async_copy(in_slice(c0 + 2), buf0, sem0)

        pltpu.make_async_copy(in_slice(c0 + 1), buf1, sem1).wait()
        compute_chunk(buf1, c0 + 1)
        return carry

    lax.fori_loop(0, _NCHUNK // 2, loop_body, 0)
    row0 = b * 2048 + r0
    pltpu.sync_copy(
        obuf, out_hbm.at[pl.ds(row0 * _OUTW, _ROWS_PER_W * _OUTW)])


@functools.cache
def _sc_call():
    return pl.kernel(
        _body,
        out_type=jax.ShapeDtypeStruct((_ROWS * _OUTW,), jnp.float32),
        mesh=plsc.VectorSubcoreMesh(
            core_axis_name="c", subcore_axis_name="s",
            num_cores=_NC, num_subcores=_NSUB,
        ),
        scratch_types=[
            pltpu.VMEM((_CHUNK, _COLS), jnp.float32),
            pltpu.VMEM((_CHUNK, _COLS), jnp.float32),
            pltpu.VMEM((_ROWS_PER_W * _OUTW,), jnp.float32),
            pltpu.SemaphoreType.DMA,
            pltpu.SemaphoreType.DMA,
        ],
        compiler_params=pltpu.CompilerParams(
            use_tc_tiling_on_sc=True, needs_layout_passes=False),
    )


@jax.jit
def kernel(inputs):
    y = _sc_call()(inputs)
    return y.reshape(8, 2048, _OUTW)
